# Initial kernel scaffold; baseline (speedup 1.0000x reference)
import jax
import jax.numpy as jnp
from jax.experimental import pallas as pl


def kernel(y_pred):
    raise NotImplementedError("write your pallas kernel here")


# speedup vs baseline: 5.5359x; 5.5359x over previous
import jax
import jax.numpy as jnp
from jax.experimental import pallas as pl
from jax.experimental.pallas import tpu as pltpu

_NC = 80
_H = 128
_W = 128
_K = 100
_BIG = 1 << 30


def _body(x_ref, out_ref, s_ref, m_ref, i_ref, d_ref):
    x = x_ref[0, :_NC, :, :]

    neg_w = jnp.full((_NC, _H, 1), -jnp.inf, jnp.float32)
    mw = jnp.maximum(
        x,
        jnp.maximum(
            jnp.concatenate([x[:, :, 1:], neg_w], axis=2),
            jnp.concatenate([neg_w, x[:, :, :-1]], axis=2),
        ),
    )
    neg_h = jnp.full((_NC, 1, _W), -jnp.inf, jnp.float32)
    hmax = jnp.maximum(
        mw,
        jnp.maximum(
            jnp.concatenate([mw[:, 1:, :], neg_h], axis=1),
            jnp.concatenate([neg_h, mw[:, :-1, :]], axis=1),
        ),
    )
    s = jnp.where(hmax == x, x, 0.0)
    s_ref[...] = s

    c_i = jax.lax.broadcasted_iota(jnp.int32, (_NC, _H, _W), 0)
    h_i = jax.lax.broadcasted_iota(jnp.int32, (_NC, _H, _W), 1)
    w_i = jax.lax.broadcasted_iota(jnp.int32, (_NC, _H, _W), 2)
    nhwc = (h_i * _W + w_i) * _NC + c_i

    blk_max = jnp.max(jnp.max(s, axis=0), axis=1, keepdims=True)
    at_max = s == blk_max[None, :, :]
    masked_idx = jnp.where(at_max, nhwc, _BIG)
    blk_idx = jnp.min(jnp.min(masked_idx, axis=0), axis=1, keepdims=True)
    m_ref[...] = blk_max
    i_ref[...] = blk_idx

    c_i2 = jax.lax.broadcasted_iota(jnp.int32, (_NC, 1, _W), 0)
    w_i2 = jax.lax.broadcasted_iota(jnp.int32, (_NC, 1, _W), 2)
    lane2 = jax.lax.broadcasted_iota(jnp.int32, (1, _W), 1)
    lane3 = jax.lax.broadcasted_iota(jnp.int32, (1, 1, _W), 2)

    def step(k, _):
        blk_m = m_ref[...]
        m = jnp.max(blk_m)
        istar = jnp.min(jnp.where(blk_m == m, i_ref[...], _BIG))
        h = istar // (_W * _NC)
        r = istar % (_W * _NC)
        w = r // _NC
        c = r % _NC

        xs = w.astype(jnp.float32)
        ys = h.astype(jnp.float32)

        def coord(ch):
            row = x_ref[0, _NC + ch, pl.ds(h, 1), :]
            return jnp.sum(jnp.where(lane2 == w, row, 0.0))

        c0, c1, c2, c3 = coord(0), coord(1), coord(2), coord(3)

        def put(f, val):
            d_ref[f, pl.ds(k, 1), :] = val[None, None]

        put(0, c.astype(jnp.float32) + 1.0)
        put(1, m)
        put(2, (4.0 * xs - c0) * (1.0 / 512.0))
        put(3, (4.0 * ys - c1) * (1.0 / 512.0))
        put(4, (4.0 * xs + c2) * (1.0 / 512.0))
        put(5, (4.0 * ys + c3) * (1.0 / 512.0))
        put(6, ys)
        put(7, xs)

        srow = s_ref[pl.ds(c, 1), pl.ds(h, 1), :]
        s_ref[pl.ds(c, 1), pl.ds(h, 1), :] = jnp.where(lane3 == w, -1.0, srow)
        slab = s_ref[:, pl.ds(h, 1), :]
        mh = jnp.max(slab)
        idx_slab = (h * _W + w_i2) * _NC + c_i2
        ih = jnp.min(jnp.where(slab == mh, idx_slab, _BIG))
        m_ref[pl.ds(h, 1), :] = mh[None, None]
        i_ref[pl.ds(h, 1), :] = ih[None, None]
        return 0

    jax.lax.fori_loop(0, _K, step, 0)
    out_ref[0, :, :] = jnp.concatenate(
        [d_ref[f, :, :] for f in range(8)], axis=1
    )


@jax.jit
def kernel(y_pred):
    out = pl.pallas_call(
        _body,
        grid=(y_pred.shape[0],),
        in_specs=[pl.BlockSpec((1, _NC + 4, _H, _W), lambda b: (b, 0, 0, 0))],
        out_specs=pl.BlockSpec((1, _K, 8), lambda b: (b, 0, 0)),
        out_shape=jax.ShapeDtypeStruct((y_pred.shape[0], _K, 8), jnp.float32),
        scratch_shapes=[
            pltpu.VMEM((_NC, _H, _W), jnp.float32),
            pltpu.VMEM((_H, 1), jnp.float32),
            pltpu.VMEM((_H, 1), jnp.int32),
            pltpu.VMEM((8, _K, 1), jnp.float32),
        ],
    )(y_pred)
    return out


# speedup vs baseline: 6.1895x; 1.1181x over previous
import jax
import jax.numpy as jnp
from jax.experimental import pallas as pl
from jax.experimental.pallas import tpu as pltpu

_NC = 80
_H = 128
_W = 128
_K = 100
_BIG = 1 << 30


def _body(x_ref, out_ref, s_ref, m_ref, i_ref, d_ref):
    x = x_ref[0, :_NC, :, :]

    neg_w = jnp.full((_NC, _H, 1), -jnp.inf, jnp.float32)
    mw = jnp.maximum(
        x,
        jnp.maximum(
            jnp.concatenate([x[:, :, 1:], neg_w], axis=2),
            jnp.concatenate([neg_w, x[:, :, :-1]], axis=2),
        ),
    )
    neg_h = jnp.full((_NC, 1, _W), -jnp.inf, jnp.float32)
    hmax = jnp.maximum(
        mw,
        jnp.maximum(
            jnp.concatenate([mw[:, 1:, :], neg_h], axis=1),
            jnp.concatenate([neg_h, mw[:, :-1, :]], axis=1),
        ),
    )
    s = jnp.where(hmax == x, x, 0.0)
    st = jnp.transpose(s, (1, 0, 2))
    s_ref[...] = st

    c_i = jax.lax.broadcasted_iota(jnp.int32, (_H, _NC, _W), 1)
    w_i = jax.lax.broadcasted_iota(jnp.int32, (_H, _NC, _W), 2)
    h_i = jax.lax.broadcasted_iota(jnp.int32, (_H, _NC, _W), 0)
    nhwc = (h_i * _W + w_i) * _NC + c_i

    blk_max = jnp.max(jnp.max(st, axis=1), axis=1, keepdims=True)
    at_max = st == blk_max[:, :, None]
    blk_idx = jnp.min(
        jnp.min(jnp.where(at_max, nhwc, _BIG), axis=1), axis=1, keepdims=True
    )
    m_ref[...] = blk_max
    i_ref[...] = blk_idx

    base2 = (
        jax.lax.broadcasted_iota(jnp.int32, (1, _NC, _W), 2) * _NC
        + jax.lax.broadcasted_iota(jnp.int32, (1, _NC, _W), 1)
    )
    lane2 = jax.lax.broadcasted_iota(jnp.int32, (1, _W), 1)
    lane3 = jax.lax.broadcasted_iota(jnp.int32, (1, 1, _W), 2)

    def step(k, _):
        blk_m = m_ref[...]
        m = jnp.max(blk_m)
        istar = jnp.min(jnp.where(blk_m == m, i_ref[...], _BIG))
        h = istar // (_W * _NC)
        r = istar % (_W * _NC)
        w = r // _NC
        c = r % _NC

        xs = w.astype(jnp.float32)
        ys = h.astype(jnp.float32)

        def coord(ch):
            row = x_ref[0, _NC + ch, pl.ds(h, 1), :]
            return jnp.sum(jnp.where(lane2 == w, row, 0.0))

        c0, c1, c2, c3 = coord(0), coord(1), coord(2), coord(3)

        def put(f, val):
            d_ref[f, pl.ds(k, 1), :] = val[None, None]

        put(0, c.astype(jnp.float32) + 1.0)
        put(1, m)
        put(2, (4.0 * xs - c0) * (1.0 / 512.0))
        put(3, (4.0 * ys - c1) * (1.0 / 512.0))
        put(4, (4.0 * xs + c2) * (1.0 / 512.0))
        put(5, (4.0 * ys + c3) * (1.0 / 512.0))
        put(6, ys)
        put(7, xs)

        srow = s_ref[pl.ds(h, 1), pl.ds(c, 1), :]
        s_ref[pl.ds(h, 1), pl.ds(c, 1), :] = jnp.where(lane3 == w, -1.0, srow)
        slab = s_ref[pl.ds(h, 1), :, :]
        mh = jnp.max(slab)
        ih = jnp.min(jnp.where(slab == mh, base2 + h * (_W * _NC), _BIG))
        m_ref[pl.ds(h, 1), :] = mh[None, None]
        i_ref[pl.ds(h, 1), :] = ih[None, None]
        return 0

    jax.lax.fori_loop(0, _K, step, 0)
    out_ref[0, :, :] = jnp.concatenate(
        [d_ref[f, :, :] for f in range(8)], axis=1
    )


@jax.jit
def kernel(y_pred):
    out = pl.pallas_call(
        _body,
        grid=(y_pred.shape[0],),
        in_specs=[pl.BlockSpec((1, _NC + 4, _H, _W), lambda b: (b, 0, 0, 0))],
        out_specs=pl.BlockSpec((1, _K, 8), lambda b: (b, 0, 0)),
        out_shape=jax.ShapeDtypeStruct((y_pred.shape[0], _K, 8), jnp.float32),
        scratch_shapes=[
            pltpu.VMEM((_H, _NC, _W), jnp.float32),
            pltpu.VMEM((_H, 1), jnp.float32),
            pltpu.VMEM((_H, 1), jnp.int32),
            pltpu.VMEM((8, _K, 1), jnp.float32),
        ],
    )(y_pred)
    return out


# speedup vs baseline: 10.9470x; 1.7686x over previous
import jax
import jax.numpy as jnp
from jax.experimental import pallas as pl
from jax.experimental.pallas import tpu as pltpu

_NC = 80
_H = 128
_W = 128
_K = 100
_BIG = 1 << 30
_NB = 2


def _body(x_ref, out_ref, s_ref, m_ref, i_ref, d_ref):
    for b in range(_NB):
        x = x_ref[b, :_NC, :, :]

        neg_w = jnp.full((_NC, _H, 1), -jnp.inf, jnp.float32)
        mw = jnp.maximum(
            x,
            jnp.maximum(
                jnp.concatenate([x[:, :, 1:], neg_w], axis=2),
                jnp.concatenate([neg_w, x[:, :, :-1]], axis=2),
            ),
        )
        neg_h = jnp.full((_NC, 1, _W), -jnp.inf, jnp.float32)
        hmax = jnp.maximum(
            mw,
            jnp.maximum(
                jnp.concatenate([mw[:, 1:, :], neg_h], axis=1),
                jnp.concatenate([neg_h, mw[:, :-1, :]], axis=1),
            ),
        )
        s = jnp.where(hmax == x, x, 0.0)
        st = jnp.transpose(s, (1, 0, 2))
        s_ref[b] = st

        c_i = jax.lax.broadcasted_iota(jnp.int32, (_H, _NC, _W), 1)
        w_i = jax.lax.broadcasted_iota(jnp.int32, (_H, _NC, _W), 2)
        h_i = jax.lax.broadcasted_iota(jnp.int32, (_H, _NC, _W), 0)
        nhwc = (h_i * _W + w_i) * _NC + c_i

        blk_max = jnp.max(jnp.max(st, axis=1), axis=1, keepdims=True)
        at_max = st == blk_max[:, :, None]
        blk_idx = jnp.min(
            jnp.min(jnp.where(at_max, nhwc, _BIG), axis=1), axis=1, keepdims=True
        )
        m_ref[b] = blk_max
        i_ref[b] = blk_idx

    base2 = (
        jax.lax.broadcasted_iota(jnp.int32, (1, _NC, _W), 2) * _NC
        + jax.lax.broadcasted_iota(jnp.int32, (1, _NC, _W), 1)
    )
    lane2 = jax.lax.broadcasted_iota(jnp.int32, (1, _W), 1)
    lane3 = jax.lax.broadcasted_iota(jnp.int32, (1, 1, _W), 2)

    def one(b, k):
        blk_m = m_ref[b]
        mv = jnp.max(blk_m, axis=0, keepdims=True)
        iv = jnp.min(
            jnp.where(blk_m == mv, i_ref[b], _BIG), axis=0, keepdims=True
        )
        istar = jnp.max(iv)
        h = istar // (_W * _NC)
        r = istar % (_W * _NC)
        w = r // _NC
        c = r % _NC

        rv = iv % (_W * _NC)
        xs = (rv // _NC).astype(jnp.float32)
        ys = (iv // (_W * _NC)).astype(jnp.float32)
        cls = (rv % _NC).astype(jnp.float32) + 1.0
        wv = rv // _NC

        def coord(ch):
            row = x_ref[b, _NC + ch, pl.ds(h, 1), :]
            return jnp.sum(jnp.where(lane2 == wv, row, 0.0), axis=1, keepdims=True)

        c0, c1, c2, c3 = coord(0), coord(1), coord(2), coord(3)

        def put(f, val):
            d_ref[b, f, pl.ds(k, 1), :] = val

        put(0, cls)
        put(1, mv)
        put(2, (4.0 * xs - c0) * (1.0 / 512.0))
        put(3, (4.0 * ys - c1) * (1.0 / 512.0))
        put(4, (4.0 * xs + c2) * (1.0 / 512.0))
        put(5, (4.0 * ys + c3) * (1.0 / 512.0))
        put(6, ys)
        put(7, xs)

        srow = s_ref[b, pl.ds(h, 1), pl.ds(c, 1), :]
        s_ref[b, pl.ds(h, 1), pl.ds(c, 1), :] = jnp.where(lane3 == w, -1.0, srow)
        slab = s_ref[b, pl.ds(h, 1), :, :]
        mh = jnp.max(
            jnp.max(slab, axis=1, keepdims=True), axis=2, keepdims=True
        )
        masked = jnp.where(slab == mh, base2 + h * (_W * _NC), _BIG)
        ih = jnp.min(
            jnp.min(masked, axis=1, keepdims=True), axis=2, keepdims=True
        )
        m_ref[b, pl.ds(h, 1), :] = mh[0]
        i_ref[b, pl.ds(h, 1), :] = ih[0]

    def step(k, _):
        for b in range(_NB):
            one(b, k)
        return 0

    jax.lax.fori_loop(0, _K, step, 0)
    for b in range(_NB):
        out_ref[b, :, :] = jnp.concatenate(
            [d_ref[b, f, :, :] for f in range(8)], axis=1
        )


@jax.jit
def kernel(y_pred):
    out = pl.pallas_call(
        _body,
        grid=(y_pred.shape[0] // _NB,),
        in_specs=[pl.BlockSpec((_NB, _NC + 4, _H, _W), lambda b: (b, 0, 0, 0))],
        out_specs=pl.BlockSpec((_NB, _K, 8), lambda b: (b, 0, 0)),
        out_shape=jax.ShapeDtypeStruct((y_pred.shape[0], _K, 8), jnp.float32),
        scratch_shapes=[
            pltpu.VMEM((_NB, _H, _NC, _W), jnp.float32),
            pltpu.VMEM((_NB, _H, 1), jnp.float32),
            pltpu.VMEM((_NB, _H, 1), jnp.int32),
            pltpu.VMEM((_NB, 8, _K, 1), jnp.float32),
        ],
    )(y_pred)
    return out


# speedup vs baseline: 13.3715x; 1.2215x over previous
import jax
import jax.numpy as jnp
from jax.experimental import pallas as pl
from jax.experimental.pallas import tpu as pltpu

_NC = 80
_H = 128
_W = 128
_K = 100
_BIG = 1 << 30
_NB = 2


def _body(x_ref, out_ref, s_ref, m_ref, i_ref, d_ref):
    for b in range(_NB):
        x = x_ref[b, :_NC, :, :]

        neg_w = jnp.full((_NC, _H, 1), -jnp.inf, jnp.float32)
        mw = jnp.maximum(
            x,
            jnp.maximum(
                jnp.concatenate([x[:, :, 1:], neg_w], axis=2),
                jnp.concatenate([neg_w, x[:, :, :-1]], axis=2),
            ),
        )
        neg_h = jnp.full((_NC, 1, _W), -jnp.inf, jnp.float32)
        hmax = jnp.maximum(
            mw,
            jnp.maximum(
                jnp.concatenate([mw[:, 1:, :], neg_h], axis=1),
                jnp.concatenate([neg_h, mw[:, :-1, :]], axis=1),
            ),
        )
        s = jnp.where(hmax == x, x, 0.0)
        st = jnp.transpose(s, (1, 0, 2))
        s_ref[b] = st

        c_i = jax.lax.broadcasted_iota(jnp.int32, (_H, _NC, _W), 1)
        w_i = jax.lax.broadcasted_iota(jnp.int32, (_H, _NC, _W), 2)
        h_i = jax.lax.broadcasted_iota(jnp.int32, (_H, _NC, _W), 0)
        nhwc = (h_i * _W + w_i) * _NC + c_i

        blk_max = jnp.max(jnp.max(st, axis=1), axis=1, keepdims=True)
        at_max = st == blk_max[:, :, None]
        blk_idx = jnp.min(
            jnp.min(jnp.where(at_max, nhwc, _BIG), axis=1), axis=1, keepdims=True
        )
        m_ref[b] = jnp.transpose(blk_max, (1, 0))
        i_ref[b] = jnp.transpose(blk_idx, (1, 0))

    base2 = (
        jax.lax.broadcasted_iota(jnp.int32, (1, _NC, _W), 2) * _NC
        + jax.lax.broadcasted_iota(jnp.int32, (1, _NC, _W), 1)
    )
    lane2 = jax.lax.broadcasted_iota(jnp.int32, (1, _W), 1)
    sub3 = jax.lax.broadcasted_iota(jnp.int32, (1, _NC, _W), 1)
    lane3 = jax.lax.broadcasted_iota(jnp.int32, (1, _NC, _W), 2)

    def one(b, k):
        mrow = m_ref[b]
        irow = i_ref[b]
        mv = jnp.max(mrow, axis=1, keepdims=True)
        iv = jnp.min(
            jnp.where(mrow == mv, irow, _BIG), axis=1, keepdims=True
        )
        istar = jnp.max(iv)
        h = istar // (_W * _NC)
        r = istar % (_W * _NC)
        w = r // _NC
        c = r % _NC

        rv = iv % (_W * _NC)
        xs = (rv // _NC).astype(jnp.float32)
        ys = (iv // (_W * _NC)).astype(jnp.float32)
        cls = (rv % _NC).astype(jnp.float32) + 1.0
        wv = rv // _NC

        def coord(ch):
            row = x_ref[b, _NC + ch, pl.ds(h, 1), :]
            return jnp.sum(jnp.where(lane2 == wv, row, 0.0), axis=1, keepdims=True)

        c0, c1, c2, c3 = coord(0), coord(1), coord(2), coord(3)

        def put(f, val):
            d_ref[b, f, pl.ds(k, 1), :] = val

        put(0, cls)
        put(1, mv)
        put(2, (4.0 * xs - c0) * (1.0 / 512.0))
        put(3, (4.0 * ys - c1) * (1.0 / 512.0))
        put(4, (4.0 * xs + c2) * (1.0 / 512.0))
        put(5, (4.0 * ys + c3) * (1.0 / 512.0))
        put(6, ys)
        put(7, xs)

        slab = s_ref[b, pl.ds(h, 1), :, :]
        slab2 = jnp.where((sub3 == c) & (lane3 == w), -1.0, slab)
        s_ref[b, pl.ds(h, 1), :, :] = slab2
        mh = jnp.max(
            jnp.max(slab2, axis=1, keepdims=True), axis=2, keepdims=True
        )
        masked = jnp.where(slab2 == mh, base2 + h * (_W * _NC), _BIG)
        ih = jnp.min(
            jnp.min(masked, axis=1, keepdims=True), axis=2, keepdims=True
        )
        m_ref[b] = jnp.where(lane2 == h, mh[0], mrow)
        i_ref[b] = jnp.where(lane2 == h, ih[0], irow)

    def step(k, _):
        for b in range(_NB):
            one(b, k)
        return 0

    jax.lax.fori_loop(0, _K, step, 0)
    for b in range(_NB):
        out_ref[b, :, :] = jnp.concatenate(
            [d_ref[b, f, :, :] for f in range(8)], axis=1
        )


@jax.jit
def kernel(y_pred):
    out = pl.pallas_call(
        _body,
        grid=(y_pred.shape[0] // _NB,),
        in_specs=[pl.BlockSpec((_NB, _NC + 4, _H, _W), lambda b: (b, 0, 0, 0))],
        out_specs=pl.BlockSpec((_NB, _K, 8), lambda b: (b, 0, 0)),
        out_shape=jax.ShapeDtypeStruct((y_pred.shape[0], _K, 8), jnp.float32),
        scratch_shapes=[
            pltpu.VMEM((_NB, _H, _NC, _W), jnp.float32),
            pltpu.VMEM((_NB, 1, _H), jnp.float32),
            pltpu.VMEM((_NB, 1, _H), jnp.int32),
            pltpu.VMEM((_NB, 8, _K, 1), jnp.float32),
        ],
    )(y_pred)
    return out


# speedup vs baseline: 20.1534x; 1.5072x over previous
import jax
import jax.numpy as jnp
from jax.experimental import pallas as pl
from jax.experimental.pallas import tpu as pltpu

_NC = 80
_H = 128
_W = 128
_K = 100
_BIG = 1 << 30
_NB = 4


def _body(x_hbm, out_ref, xv_ref, s_ref, m_ref, i_ref, d_ref, sem):
    g = pl.program_id(0)
    cp = pltpu.make_async_copy(x_hbm.at[pl.ds(g * _NB, _NB)], xv_ref, sem)
    cp.start()
    cp.wait()
    x_ref = xv_ref
    for b in range(_NB):
        x = x_ref[b, :_NC, :, :]

        neg_w = jnp.full((_NC, _H, 1), -jnp.inf, jnp.float32)
        mw = jnp.maximum(
            x,
            jnp.maximum(
                jnp.concatenate([x[:, :, 1:], neg_w], axis=2),
                jnp.concatenate([neg_w, x[:, :, :-1]], axis=2),
            ),
        )
        neg_h = jnp.full((_NC, 1, _W), -jnp.inf, jnp.float32)
        hmax = jnp.maximum(
            mw,
            jnp.maximum(
                jnp.concatenate([mw[:, 1:, :], neg_h], axis=1),
                jnp.concatenate([neg_h, mw[:, :-1, :]], axis=1),
            ),
        )
        s = jnp.where(hmax == x, x, 0.0)
        st = jnp.transpose(s, (1, 0, 2))
        s_ref[b] = st

        c_i = jax.lax.broadcasted_iota(jnp.int32, (_H, _NC, _W), 1)
        w_i = jax.lax.broadcasted_iota(jnp.int32, (_H, _NC, _W), 2)
        h_i = jax.lax.broadcasted_iota(jnp.int32, (_H, _NC, _W), 0)
        nhwc = (h_i * _W + w_i) * _NC + c_i

        blk_max = jnp.max(jnp.max(st, axis=1), axis=1, keepdims=True)
        at_max = st == blk_max[:, :, None]
        blk_idx = jnp.min(
            jnp.min(jnp.where(at_max, nhwc, _BIG), axis=1), axis=1, keepdims=True
        )
        m_ref[b] = jnp.transpose(blk_max, (1, 0))
        i_ref[b] = jnp.transpose(blk_idx, (1, 0))

    base2 = (
        jax.lax.broadcasted_iota(jnp.int32, (1, _NC, _W), 2) * _NC
        + jax.lax.broadcasted_iota(jnp.int32, (1, _NC, _W), 1)
    )
    lane2 = jax.lax.broadcasted_iota(jnp.int32, (1, _W), 1)
    sub3 = jax.lax.broadcasted_iota(jnp.int32, (1, _NC, _W), 1)
    lane3 = jax.lax.broadcasted_iota(jnp.int32, (1, _NC, _W), 2)

    def one(b, k):
        mrow = m_ref[b]
        irow = i_ref[b]
        mv = jnp.max(mrow, axis=1, keepdims=True)
        iv = jnp.min(
            jnp.where(mrow == mv, irow, _BIG), axis=1, keepdims=True
        )
        istar = jnp.max(iv)
        h = istar // (_W * _NC)
        r = istar % (_W * _NC)
        w = r // _NC
        c = r % _NC

        rv = iv % (_W * _NC)
        xs = (rv // _NC).astype(jnp.float32)
        ys = (iv // (_W * _NC)).astype(jnp.float32)
        cls = (rv % _NC).astype(jnp.float32) + 1.0
        wv = rv // _NC

        def coord(ch):
            row = x_ref[b, _NC + ch, pl.ds(h, 1), :]
            return jnp.sum(jnp.where(lane2 == wv, row, 0.0), axis=1, keepdims=True)

        c0, c1, c2, c3 = coord(0), coord(1), coord(2), coord(3)

        def put(f, val):
            d_ref[b, f, pl.ds(k, 1), :] = val

        put(0, cls)
        put(1, mv)
        put(2, (4.0 * xs - c0) * (1.0 / 512.0))
        put(3, (4.0 * ys - c1) * (1.0 / 512.0))
        put(4, (4.0 * xs + c2) * (1.0 / 512.0))
        put(5, (4.0 * ys + c3) * (1.0 / 512.0))
        put(6, ys)
        put(7, xs)

        slab = s_ref[b, pl.ds(h, 1), :, :]
        slab2 = jnp.where((sub3 == c) & (lane3 == w), -1.0, slab)
        s_ref[b, pl.ds(h, 1), :, :] = slab2
        mh = jnp.max(
            jnp.max(slab2, axis=1, keepdims=True), axis=2, keepdims=True
        )
        masked = jnp.where(slab2 == mh, base2 + h * (_W * _NC), _BIG)
        ih = jnp.min(
            jnp.min(masked, axis=1, keepdims=True), axis=2, keepdims=True
        )
        m_ref[b] = jnp.where(lane2 == h, mh[0], mrow)
        i_ref[b] = jnp.where(lane2 == h, ih[0], irow)

    def step(k, _):
        for b in range(_NB):
            one(b, k)
        return 0

    jax.lax.fori_loop(0, _K, step, 0)
    for b in range(_NB):
        out_ref[b, :, :] = jnp.concatenate(
            [d_ref[b, f, :, :] for f in range(8)], axis=1
        )


@jax.jit
def kernel(y_pred):
    out = pl.pallas_call(
        _body,
        grid=(y_pred.shape[0] // _NB,),
        in_specs=[pl.BlockSpec(memory_space=pl.ANY)],
        out_specs=pl.BlockSpec((_NB, _K, 8), lambda b: (b, 0, 0)),
        out_shape=jax.ShapeDtypeStruct((y_pred.shape[0], _K, 8), jnp.float32),
        scratch_shapes=[
            pltpu.VMEM((_NB, _NC + 4, _H, _W), jnp.float32),
            pltpu.VMEM((_NB, _H, _NC, _W), jnp.float32),
            pltpu.VMEM((_NB, 1, _H), jnp.float32),
            pltpu.VMEM((_NB, 1, _H), jnp.int32),
            pltpu.VMEM((_NB, 8, _K, 1), jnp.float32),
            pltpu.SemaphoreType.DMA,
        ],
    )(y_pred)
    return out


# speedup vs baseline: 26.4977x; 1.3148x over previous
import jax
import jax.numpy as jnp
from jax.experimental import pallas as pl
from jax.experimental.pallas import tpu as pltpu

_NC = 80
_H = 128
_W = 128
_K = 100
_BIG = 1 << 30
_NB = 4


def _body(x_hbm, out_ref, xv_ref, s_ref, m_ref, i_ref, d_ref, sem):
    g = pl.program_id(0)
    cp = pltpu.make_async_copy(x_hbm.at[pl.ds(g * _NB, _NB)], xv_ref, sem)
    cp.start()
    cp.wait()
    x_ref = xv_ref
    mv0 = []
    iv0 = []
    for b in range(_NB):
        x = x_ref[b, :_NC, :, :]

        neg_w = jnp.full((_NC, _H, 1), -jnp.inf, jnp.float32)
        mw = jnp.maximum(
            x,
            jnp.maximum(
                jnp.concatenate([x[:, :, 1:], neg_w], axis=2),
                jnp.concatenate([neg_w, x[:, :, :-1]], axis=2),
            ),
        )
        neg_h = jnp.full((_NC, 1, _W), -jnp.inf, jnp.float32)
        hmax = jnp.maximum(
            mw,
            jnp.maximum(
                jnp.concatenate([mw[:, 1:, :], neg_h], axis=1),
                jnp.concatenate([neg_h, mw[:, :-1, :]], axis=1),
            ),
        )
        s = jnp.where(hmax == x, x, 0.0)
        st = jnp.transpose(s, (1, 0, 2))
        s_ref[b] = st

        c_i = jax.lax.broadcasted_iota(jnp.int32, (_H, _NC, _W), 1)
        w_i = jax.lax.broadcasted_iota(jnp.int32, (_H, _NC, _W), 2)
        h_i = jax.lax.broadcasted_iota(jnp.int32, (_H, _NC, _W), 0)
        nhwc = (h_i * _W + w_i) * _NC + c_i

        blk_max = jnp.max(jnp.max(st, axis=1), axis=1, keepdims=True)
        at_max = st == blk_max[:, :, None]
        blk_idx = jnp.min(
            jnp.min(jnp.where(at_max, nhwc, _BIG), axis=1), axis=1, keepdims=True
        )
        mr = jnp.transpose(blk_max, (1, 0))
        ir = jnp.transpose(blk_idx, (1, 0))
        m_ref[b] = mr
        i_ref[b] = ir
        mv0.append(jnp.max(mr, axis=1, keepdims=True))
        iv0.append(
            jnp.min(
                jnp.where(mr == mv0[b], ir, _BIG), axis=1, keepdims=True
            )
        )

    base2 = (
        jax.lax.broadcasted_iota(jnp.int32, (1, _NC, _W), 2) * _NC
        + jax.lax.broadcasted_iota(jnp.int32, (1, _NC, _W), 1)
    )
    lane2 = jax.lax.broadcasted_iota(jnp.int32, (1, _W), 1)
    sub3 = jax.lax.broadcasted_iota(jnp.int32, (1, _NC, _W), 1)
    lane3 = jax.lax.broadcasted_iota(jnp.int32, (1, _NC, _W), 2)

    def one(b, k, mv, iv):
        istar = jnp.max(iv)
        h = istar // (_W * _NC)
        r = istar % (_W * _NC)
        w = r // _NC
        c = r % _NC

        rv = iv % (_W * _NC)
        xs = (rv // _NC).astype(jnp.float32)
        ys = (iv // (_W * _NC)).astype(jnp.float32)
        cls = (rv % _NC).astype(jnp.float32) + 1.0
        wv = rv // _NC

        def coord(ch):
            row = x_ref[b, _NC + ch, pl.ds(h, 1), :]
            return jnp.sum(jnp.where(lane2 == wv, row, 0.0), axis=1, keepdims=True)

        c0, c1, c2, c3 = coord(0), coord(1), coord(2), coord(3)

        def put(f, val):
            d_ref[b, f, pl.ds(k, 1), :] = val

        put(0, cls)
        put(1, mv)
        put(2, (4.0 * xs - c0) * (1.0 / 512.0))
        put(3, (4.0 * ys - c1) * (1.0 / 512.0))
        put(4, (4.0 * xs + c2) * (1.0 / 512.0))
        put(5, (4.0 * ys + c3) * (1.0 / 512.0))
        put(6, ys)
        put(7, xs)

        mrow = m_ref[b]
        irow = i_ref[b]
        not_h = lane2 != h
        runner_m = jnp.max(
            jnp.where(not_h, mrow, -1.0), axis=1, keepdims=True
        )
        runner_i = jnp.min(
            jnp.where((mrow == runner_m) & not_h, irow, _BIG),
            axis=1,
            keepdims=True,
        )

        slab = s_ref[b, pl.ds(h, 1), :, :]
        slab2 = jnp.where((sub3 == c) & (lane3 == w), -1.0, slab)
        s_ref[b, pl.ds(h, 1), :, :] = slab2
        mh = jnp.max(
            jnp.max(slab2, axis=1, keepdims=True), axis=2, keepdims=True
        )
        masked = jnp.where(slab2 == mh, base2 + h * (_W * _NC), _BIG)
        ih = jnp.min(
            jnp.min(masked, axis=1, keepdims=True), axis=2, keepdims=True
        )
        m_ref[b] = jnp.where(lane2 == h, mh[0], mrow)
        i_ref[b] = jnp.where(lane2 == h, ih[0], irow)

        nm = jnp.maximum(mh[0], runner_m)
        ni = jnp.minimum(
            jnp.where(mh[0] == nm, ih[0], _BIG),
            jnp.where(runner_m == nm, runner_i, _BIG),
        )
        return nm, ni

    def step(k, carry):
        out = []
        for b in range(_NB):
            out.append(one(b, k, carry[2 * b], carry[2 * b + 1]))
        return tuple(x for p in out for x in p)

    init = tuple(x for b in range(_NB) for x in (mv0[b], iv0[b]))
    jax.lax.fori_loop(0, _K, step, init)
    for b in range(_NB):
        out_ref[b, :, :] = jnp.concatenate(
            [d_ref[b, f, :, :] for f in range(8)], axis=1
        )


@jax.jit
def kernel(y_pred):
    out = pl.pallas_call(
        _body,
        grid=(y_pred.shape[0] // _NB,),
        in_specs=[pl.BlockSpec(memory_space=pl.ANY)],
        out_specs=pl.BlockSpec((_NB, _K, 8), lambda b: (b, 0, 0)),
        out_shape=jax.ShapeDtypeStruct((y_pred.shape[0], _K, 8), jnp.float32),
        scratch_shapes=[
            pltpu.VMEM((_NB, _NC + 4, _H, _W), jnp.float32),
            pltpu.VMEM((_NB, _H, _NC, _W), jnp.float32),
            pltpu.VMEM((_NB, 1, _H), jnp.float32),
            pltpu.VMEM((_NB, 1, _H), jnp.int32),
            pltpu.VMEM((_NB, 8, _K, 1), jnp.float32),
            pltpu.SemaphoreType.DMA,
        ],
    )(y_pred)
    return out
